# baseline (speedup 1.0000x reference)
import functools

import jax
import jax.numpy as jnp
from jax import lax
from jax.experimental import pallas as pl
from jax.experimental.pallas import tpu as pltpu
from jax.experimental.pallas import tpu_sc as plsc

B = 128
N = 32768
K = 30
NCHUNK = N // 16
UNROLL = 4
NGROUP = NCHUNK // UNROLL
CAP = 4096
EPS = 1e-07
LN2 = 0.6931471805599453
NEG_BIG = -3.4e38


def _any(m):
    return plsc.all_reduce_population_count(m)[0] > 0


def _sort_asc(v):
    return plsc.sort_key_val(v, v)[0]


def _sort_desc(v):
    return plsc.sort_key_val(v, v, descending=True)[0]


def _lane(vec, i):
    li = lax.iota(jnp.int32, 16)
    return jnp.max(jnp.where(li == i, vec, NEG_BIG))


def _merge_chunk(v, state):
    H, L, th = state
    hit = _any(v > th)

    def merge(c):
        H, L, _ = c
        vs = _sort_asc(v)
        up = jnp.maximum(vs, L)
        upd = _sort_desc(up)
        nH = _sort_asc(jnp.maximum(H, upd))
        nL = _sort_desc(jnp.minimum(H, upd))
        return (nH, nL, jnp.min(nL))

    return lax.cond(hit, merge, lambda c: c, (H, L, th))


def _hl_init(c0, c1):
    h0 = _sort_asc(c0)
    l0 = _sort_desc(c1)
    H = _sort_asc(jnp.maximum(h0, l0))
    L = _sort_desc(jnp.minimum(h0, l0))
    return (H, L, jnp.min(L))


def _hl_fin(st):
    H, L, _ = st
    tv = _lane(L, K - 1 - 16)
    cgt = (jnp.sum((H > tv).astype(jnp.int32))
           + jnp.sum((L > tv).astype(jnp.int32)))
    return tv, K - cgt


def _collect(v, ivec, th, cur, cref):
    m = v >= th
    im = m.astype(jnp.int32)
    pos = cur + plsc.cumsum(im) - im
    posc = jnp.minimum(pos, CAP - 1)
    plsc.store_scatter(cref, [posc], ivec, mask=m)
    return cur + plsc.all_reduce_population_count(m)


def _scan_collect2(tb, lb, candT, candL):
    li = lax.iota(jnp.int32, 16)
    zi = jnp.zeros((16,), jnp.int32)

    t0 = tb[pl.ds(0, 16)]
    t1 = tb[pl.ds(16, 16)]
    x0 = lb[pl.ds(0, 16)]
    x1 = lb[pl.ds(16, 16)]
    curT = _collect(t0, li, NEG_BIG, zi, candT)
    curT = _collect(t1, li + 16, NEG_BIG, curT, candT)
    curL = _collect(x0, li, NEG_BIG, zi, candL)
    curL = _collect(x1, li + 16, NEG_BIG, curL, candL)
    At, Bt = t0, t1
    Al, Bl = x0, x1
    tht = jnp.min(jnp.minimum(At, Bt))
    thl = jnp.min(jnp.minimum(Al, Bl))
    for j in (2, 3):
        v = tb[pl.ds(16 * j, 16)]
        curT = _collect(v, li + 16 * j, tht, curT, candT)
        Bt = jnp.maximum(Bt, jnp.minimum(At, v))
        At = jnp.maximum(At, v)
        x = lb[pl.ds(16 * j, 16)]
        curL = _collect(x, li + 16 * j, thl, curL, candL)
        Bl = jnp.maximum(Bl, jnp.minimum(Al, x))
        Al = jnp.maximum(Al, x)
    tht = jnp.min(jnp.minimum(At, Bt))
    thl = jnp.min(jnp.minimum(Al, Bl))

    false = jnp.bool_(False)

    def drain(c, pg):
        At, Bt, tht, Al, Bl, thl, curT, curL = c
        pbase = pg * (16 * UNROLL)
        ts = [tb[pl.ds(pbase + 16 * j, 16)] for j in range(UNROLL)]
        xs = [lb[pl.ds(pbase + 16 * j, 16)] for j in range(UNROLL)]
        tmax = jnp.maximum(jnp.maximum(ts[0], ts[1]), jnp.maximum(ts[2], ts[3]))
        xmax = jnp.maximum(jnp.maximum(xs[0], xs[1]), jnp.maximum(xs[2], xs[3]))
        nBt = jnp.maximum(Bt, jnp.minimum(At, tmax))
        nAt = jnp.maximum(At, tmax)
        nBl = jnp.maximum(Bl, jnp.minimum(Al, xmax))
        nAl = jnp.maximum(Al, xmax)
        for j in range(UNROLL):
            curT = _collect(ts[j], li + (pbase + 16 * j), tht, curT, candT)
        for j in range(UNROLL):
            curL = _collect(xs[j], li + (pbase + 16 * j), thl, curL, candL)
        return (nAt, nBt, jnp.min(jnp.minimum(nAt, nBt)),
                nAl, nBl, jnp.min(jnp.minimum(nAl, nBl)),
                curT, curL)

    def it(g, carry):
        hitp, c = carry
        base = g * (16 * UNROLL)
        ts = [tb[pl.ds(base + 16 * j, 16)] for j in range(UNROLL)]
        xs = [lb[pl.ds(base + 16 * j, 16)] for j in range(UNROLL)]
        tmax = jnp.maximum(jnp.maximum(ts[0], ts[1]), jnp.maximum(ts[2], ts[3]))
        xmax = jnp.maximum(jnp.maximum(xs[0], xs[1]), jnp.maximum(xs[2], xs[3]))
        hit = _any((tmax >= c[2]) | (xmax >= c[5]))
        c = lax.cond(hitp, lambda cc: drain(cc, g - 1), lambda cc: cc, c)
        return (hit, c)

    hitp, carry = lax.fori_loop(
        1, NGROUP, it,
        (false, (At, Bt, tht, Al, Bl, thl, curT, curL)))
    carry = lax.cond(hitp, lambda cc: drain(cc, NGROUP - 1),
                     lambda cc: cc, carry)
    return jnp.max(carry[6]), jnp.max(carry[7])


def _select30(cref, cn, buf):
    li = lax.iota(jnp.int32, 16)
    minf = jnp.float32(float("-inf"))

    def gather(i):
        idxv = cref[pl.ds(i * 16, 16)]
        idxg = jnp.minimum(jnp.maximum(idxv, 0), N - 1)
        return plsc.load_gather(buf, [idxg])

    st = _hl_init(gather(0), gather(1))
    nch = (cn + 15) // 16

    def it(i, st):
        vals = gather(i)
        valid = (li + i * 16) < cn
        v = jnp.where(valid, vals, minf)
        return _merge_chunk(v, st)

    st = lax.fori_loop(2, nch, it, st)
    return _hl_fin(st)


def _scan_topk2_full(tb, lb):
    st_t = _hl_init(tb[pl.ds(0, 16)], tb[pl.ds(16, 16)])
    st_l = _hl_init(lb[pl.ds(0, 16)], lb[pl.ds(16, 16)])
    for j in (2, 3):
        st_t = _merge_chunk(tb[pl.ds(16 * j, 16)], st_t)
        st_l = _merge_chunk(lb[pl.ds(16 * j, 16)], st_l)

    def it(g, carry):
        st_t, st_l = carry
        base = g * (16 * UNROLL)
        ts = [tb[pl.ds(base + 16 * j, 16)] for j in range(UNROLL)]
        xs = [lb[pl.ds(base + 16 * j, 16)] for j in range(UNROLL)]
        tmax = jnp.maximum(jnp.maximum(ts[0], ts[1]), jnp.maximum(ts[2], ts[3]))
        xmax = jnp.maximum(jnp.maximum(xs[0], xs[1]), jnp.maximum(xs[2], xs[3]))
        hit = _any((tmax > st_t[2]) | (xmax > st_l[2]))

        def slow(c):
            st_t, st_l = c
            for j in range(UNROLL):
                st_t = _merge_chunk(ts[j], st_t)
            for j in range(UNROLL):
                st_l = _merge_chunk(xs[j], st_l)
            return (st_t, st_l)

        return lax.cond(hit, slow, lambda c: c, carry)

    st_t, st_l = lax.fori_loop(1, NGROUP, it, (st_t, st_l))
    tvt, needt = _hl_fin(st_t)
    tvl, needl = _hl_fin(st_l)
    return tvt, needt, tvl, needl


def _cand_members(cref, cn, buf, tv, need, mref):
    li = lax.iota(jnp.int32, 16)
    zi = jnp.zeros((16,), jnp.int32)
    nch = (cn + 15) // 16

    def it(i, carry):
        tie, cur = carry
        idxv = cref[pl.ds(i * 16, 16)]
        idxg = jnp.minimum(jnp.maximum(idxv, 0), N - 1)
        vals = plsc.load_gather(buf, [idxg])
        valid = (li + i * 16) < cn
        mg = valid & (vals > tv)
        me = valid & (vals == tv)
        ime = me.astype(jnp.int32)
        pe = plsc.cumsum(ime) - ime
        mm = mg | (me & (tie + pe < need))
        imm = mm.astype(jnp.int32)
        pos = cur + plsc.cumsum(imm) - imm
        plsc.store_scatter(mref, [pos], idxv, mask=mm)
        return (tie + jnp.sum(ime), cur + plsc.all_reduce_population_count(mm))

    lax.fori_loop(0, nch, it, (jnp.int32(0), zi))


def _membership_full(tb, lb, tvt, needt, tvl, needl, tmem):
    li = lax.iota(jnp.int32, 16)
    zero = jnp.int32(0)
    zi = jnp.zeros((16,), jnp.int32)

    def chunk(t, x, iv, c):
        tieT, tieL, cur, ovv = c
        mTg = t > tvt
        mTe = t == tvt
        mLg = x > tvl
        mLe = x == tvl
        iTe = mTe.astype(jnp.int32)
        iLe = mLe.astype(jnp.int32)
        peT = plsc.cumsum(iTe) - iTe
        peL = plsc.cumsum(iLe) - iLe
        memT = mTg | (mTe & (tieT + peT < needt))
        memL = mLg | (mLe & (tieL + peL < needl))
        imT = memT.astype(jnp.int32)
        pos = cur + plsc.cumsum(imT) - imT
        plsc.store_scatter(tmem, [pos], iv, mask=memT)
        return (tieT + jnp.sum(iTe),
                tieL + jnp.sum(iLe),
                cur + plsc.all_reduce_population_count(memT),
                ovv + (memT & memL).astype(jnp.int32))

    def it(g, carry):
        base = g * (16 * UNROLL)
        ts = [tb[pl.ds(base + 16 * j, 16)] for j in range(UNROLL)]
        xs = [lb[pl.ds(base + 16 * j, 16)] for j in range(UNROLL)]
        tmax = jnp.maximum(jnp.maximum(ts[0], ts[1]), jnp.maximum(ts[2], ts[3]))
        xmax = jnp.maximum(jnp.maximum(xs[0], xs[1]), jnp.maximum(xs[2], xs[3]))
        hit = _any((tmax >= tvt) | (xmax >= tvl))

        def slow(c):
            for j in range(UNROLL):
                c = chunk(ts[j], xs[j], li + (base + 16 * j), c)
            return c

        return lax.cond(hit, slow, lambda c: c, carry)

    carry = lax.fori_loop(0, NGROUP, it, (zero, zero, zi, zi))
    return jnp.sum(carry[3])


def _neg_log_sigmoid(x):
    s = 1.0 / (1.0 + jnp.exp(-x))
    y = s + jnp.float32(EPS)
    bits = plsc.bitcast(y, jnp.int32)
    e = (bits >> 23) - 127
    m = plsc.bitcast((bits & 0x7FFFFF) | 0x3F800000, jnp.float32)
    z = (m - 1.0) / (m + 1.0)
    z2 = z * z
    p = 1.0 + z2 * (jnp.float32(1 / 3) + z2 * (jnp.float32(1 / 5)
          + z2 * (jnp.float32(1 / 7) + z2 * jnp.float32(1 / 9))))
    lny = e.astype(jnp.float32) * jnp.float32(LN2) + 2.0 * z * p
    return -lny


def _sc_body(logits_hbm, targets_hbm, out_hbm,
             tbuf, lbuf, candT, candL, tmem, lmem, obuf, semt, seml):
    cid = lax.axis_index("c")
    sid = lax.axis_index("s")
    wid = sid * 2 + cid

    li = lax.iota(jnp.int32, 16)

    def row_it(r, lossvec):
        row = wid * 4 + r
        ct = pltpu.async_copy(targets_hbm.at[row], tbuf, semt)
        cl = pltpu.async_copy(logits_hbm.at[row], lbuf, seml)
        ct.wait()
        cl.wait()
        tmem[pl.ds(0, 16)] = jnp.full((16,), -1, jnp.int32)
        tmem[pl.ds(16, 16)] = jnp.full((16,), -1, jnp.int32)
        lmem[pl.ds(0, 16)] = jnp.full((16,), -2, jnp.int32)
        lmem[pl.ds(16, 16)] = jnp.full((16,), -2, jnp.int32)

        cnt, cnl = _scan_collect2(tbuf, lbuf, candT, candL)
        overflow = (cnt > CAP - 1) | (cnl > CAP - 1)

        def fast(_):
            tvt, needt = _select30(candT, cnt, tbuf)
            tvl, needl = _select30(candL, cnl, lbuf)
            _cand_members(candT, cnt, tbuf, tvt, needt, tmem)
            _cand_members(candL, cnl, lbuf, tvl, needl, lmem)
            t0 = tmem[pl.ds(0, 16)]
            t1 = tmem[pl.ds(16, 16)]
            acc = jnp.zeros((16,), jnp.int32)
            for sh in range(16):
                perm = (li + sh) & 15
                r0 = plsc.load_gather(lmem, [perm])
                r1 = plsc.load_gather(lmem, [perm + 16])
                acc = (acc + (t0 == r0).astype(jnp.int32)
                       + (t0 == r1).astype(jnp.int32)
                       + (t1 == r0).astype(jnp.int32)
                       + (t1 == r1).astype(jnp.int32))
            return jnp.sum(acc)

        def slowfb(_):
            tvt, needt, tvl, needl = _scan_topk2_full(tbuf, lbuf)
            return _membership_full(tbuf, lbuf, tvt, needt, tvl, needl, tmem)

        ov = lax.cond(overflow, slowfb, fast, None)

        t0 = jnp.maximum(tmem[pl.ds(0, 16)], 0)
        t1 = jnp.maximum(tmem[pl.ds(16, 16)], 0)
        g0 = plsc.load_gather(lbuf, [t0])
        g1 = plsc.load_gather(lbuf, [t1])
        f0 = _neg_log_sigmoid(g0)
        f1 = jnp.where(li < K - 16, _neg_log_sigmoid(g1), 0.0)
        fsum = jnp.sum(f0 + f1)
        w = 1.0 - ov.astype(jnp.float32) * jnp.float32(1.0 / K)
        loss_r = fsum * jnp.float32(1.0 / K) * w
        return jnp.where(li == r, loss_r, lossvec)

    lossvec = lax.fori_loop(0, 4, row_it, jnp.zeros((16,), jnp.float32))
    obuf[...] = lossvec
    pltpu.sync_copy(obuf, out_hbm.at[wid])


@jax.jit
def _sc_call(logits, targets):
    fn = functools.partial(
        pl.kernel,
        out_type=jax.ShapeDtypeStruct((32, 16), jnp.float32),
        mesh=plsc.VectorSubcoreMesh(core_axis_name="c", subcore_axis_name="s"),
        compiler_params=pltpu.CompilerParams(needs_layout_passes=False),
        scratch_types=[
            pltpu.VMEM((N,), jnp.float32),
            pltpu.VMEM((N,), jnp.float32),
            pltpu.VMEM((CAP,), jnp.int32),
            pltpu.VMEM((CAP,), jnp.int32),
            pltpu.VMEM((32,), jnp.int32),
            pltpu.VMEM((32,), jnp.int32),
            pltpu.VMEM((16,), jnp.float32),
            pltpu.SemaphoreType.DMA,
            pltpu.SemaphoreType.DMA,
        ],
    )(_sc_body)
    part = fn(logits, targets)
    return jnp.sum(part) * jnp.float32(1.0 / B)


def kernel(logits, targets):
    return _sc_call(logits, targets)


# speedup vs baseline: 1.0097x; 1.0097x over previous
import functools

import jax
import jax.numpy as jnp
from jax import lax
from jax.experimental import pallas as pl
from jax.experimental.pallas import tpu as pltpu
from jax.experimental.pallas import tpu_sc as plsc

B = 128
N = 32768
K = 30
NCHUNK = N // 16
UNROLL = 4
NGROUP = NCHUNK // UNROLL
CAP = 4096
EPS = 1e-07
LN2 = 0.6931471805599453
NEG_BIG = -3.4e38


def _any(m):
    return plsc.all_reduce_population_count(m)[0] > 0


def _sort_asc(v):
    return plsc.sort_key_val(v, v)[0]


def _sort_desc(v):
    return plsc.sort_key_val(v, v, descending=True)[0]


def _lane(vec, i):
    li = lax.iota(jnp.int32, 16)
    return jnp.max(jnp.where(li == i, vec, NEG_BIG))


def _merge_chunk(v, state):
    H, L, th = state
    hit = _any(v > th)

    def merge(c):
        H, L, _ = c
        vs = _sort_asc(v)
        up = jnp.maximum(vs, L)
        upd = _sort_desc(up)
        nH = _sort_asc(jnp.maximum(H, upd))
        nL = _sort_desc(jnp.minimum(H, upd))
        return (nH, nL, jnp.min(nL))

    return lax.cond(hit, merge, lambda c: c, (H, L, th))


def _hl_init(c0, c1):
    h0 = _sort_asc(c0)
    l0 = _sort_desc(c1)
    H = _sort_asc(jnp.maximum(h0, l0))
    L = _sort_desc(jnp.minimum(h0, l0))
    return (H, L, jnp.min(L))


def _hl_fin(st):
    H, L, _ = st
    tv = _lane(L, K - 1 - 16)
    cgt = (jnp.sum((H > tv).astype(jnp.int32))
           + jnp.sum((L > tv).astype(jnp.int32)))
    return tv, K - cgt


def _collect(v, ivec, th, cur, cref):
    m = v >= th
    im = m.astype(jnp.int32)
    pos = cur + plsc.cumsum(im) - im
    posc = jnp.minimum(pos, CAP - 1)
    plsc.store_scatter(cref, [posc], ivec, mask=m)
    return cur + plsc.all_reduce_population_count(m)


def _scan_collect2(tb, lb, candT, candL):
    li = lax.iota(jnp.int32, 16)
    zi = jnp.zeros((16,), jnp.int32)

    t0 = tb[pl.ds(0, 16)]
    t1 = tb[pl.ds(16, 16)]
    x0 = lb[pl.ds(0, 16)]
    x1 = lb[pl.ds(16, 16)]
    curT = _collect(t0, li, NEG_BIG, zi, candT)
    curT = _collect(t1, li + 16, NEG_BIG, curT, candT)
    curL = _collect(x0, li, NEG_BIG, zi, candL)
    curL = _collect(x1, li + 16, NEG_BIG, curL, candL)
    At, Bt = t0, t1
    Al, Bl = x0, x1
    tht = jnp.min(jnp.minimum(At, Bt))
    thl = jnp.min(jnp.minimum(Al, Bl))
    for j in (2, 3):
        v = tb[pl.ds(16 * j, 16)]
        curT = _collect(v, li + 16 * j, tht, curT, candT)
        Bt = jnp.maximum(Bt, jnp.minimum(At, v))
        At = jnp.maximum(At, v)
        x = lb[pl.ds(16 * j, 16)]
        curL = _collect(x, li + 16 * j, thl, curL, candL)
        Bl = jnp.maximum(Bl, jnp.minimum(Al, x))
        Al = jnp.maximum(Al, x)
    tht = jnp.min(jnp.minimum(At, Bt))
    thl = jnp.min(jnp.minimum(Al, Bl))

    def it(g, carry):
        At, Bt, tht, Al, Bl, thl, curT, curL = carry
        base = g * (16 * UNROLL)
        ts = [tb[pl.ds(base + 16 * j, 16)] for j in range(UNROLL)]
        xs = [lb[pl.ds(base + 16 * j, 16)] for j in range(UNROLL)]
        tmax = jnp.maximum(jnp.maximum(ts[0], ts[1]), jnp.maximum(ts[2], ts[3]))
        xmax = jnp.maximum(jnp.maximum(xs[0], xs[1]), jnp.maximum(xs[2], xs[3]))
        hit = _any((tmax >= tht) | (xmax >= thl))

        def slow(c):
            At, Bt, tht, Al, Bl, thl, curT, curL = c
            nBt = jnp.maximum(Bt, jnp.minimum(At, tmax))
            nAt = jnp.maximum(At, tmax)
            nBl = jnp.maximum(Bl, jnp.minimum(Al, xmax))
            nAl = jnp.maximum(Al, xmax)
            for j in range(UNROLL):
                curT = _collect(ts[j], li + (base + 16 * j), tht, curT, candT)
            for j in range(UNROLL):
                curL = _collect(xs[j], li + (base + 16 * j), thl, curL, candL)
            return (nAt, nBt, jnp.min(jnp.minimum(nAt, nBt)),
                    nAl, nBl, jnp.min(jnp.minimum(nAl, nBl)),
                    curT, curL)

        return lax.cond(hit, slow, lambda c: c, carry)

    carry = lax.fori_loop(1, NGROUP, it,
                          (At, Bt, tht, Al, Bl, thl, curT, curL))
    return jnp.max(carry[6]), jnp.max(carry[7])


def _select30(cref, cn, buf):
    li = lax.iota(jnp.int32, 16)
    minf = jnp.float32(float("-inf"))

    def gather(i):
        idxv = cref[pl.ds(i * 16, 16)]
        idxg = jnp.minimum(jnp.maximum(idxv, 0), N - 1)
        return plsc.load_gather(buf, [idxg])

    st = _hl_init(gather(0), gather(1))
    nch = (cn + 15) // 16

    def it(i, st):
        vals = gather(i)
        valid = (li + i * 16) < cn
        v = jnp.where(valid, vals, minf)
        return _merge_chunk(v, st)

    st = lax.fori_loop(2, nch, it, st)
    return _hl_fin(st)


def _scan_topk2_full(tb, lb):
    st_t = _hl_init(tb[pl.ds(0, 16)], tb[pl.ds(16, 16)])
    st_l = _hl_init(lb[pl.ds(0, 16)], lb[pl.ds(16, 16)])
    for j in (2, 3):
        st_t = _merge_chunk(tb[pl.ds(16 * j, 16)], st_t)
        st_l = _merge_chunk(lb[pl.ds(16 * j, 16)], st_l)

    def it(g, carry):
        st_t, st_l = carry
        base = g * (16 * UNROLL)
        ts = [tb[pl.ds(base + 16 * j, 16)] for j in range(UNROLL)]
        xs = [lb[pl.ds(base + 16 * j, 16)] for j in range(UNROLL)]
        tmax = jnp.maximum(jnp.maximum(ts[0], ts[1]), jnp.maximum(ts[2], ts[3]))
        xmax = jnp.maximum(jnp.maximum(xs[0], xs[1]), jnp.maximum(xs[2], xs[3]))
        hit = _any((tmax > st_t[2]) | (xmax > st_l[2]))

        def slow(c):
            st_t, st_l = c
            for j in range(UNROLL):
                st_t = _merge_chunk(ts[j], st_t)
            for j in range(UNROLL):
                st_l = _merge_chunk(xs[j], st_l)
            return (st_t, st_l)

        return lax.cond(hit, slow, lambda c: c, carry)

    st_t, st_l = lax.fori_loop(1, NGROUP, it, (st_t, st_l))
    tvt, needt = _hl_fin(st_t)
    tvl, needl = _hl_fin(st_l)
    return tvt, needt, tvl, needl


def _cand_members(cref, cn, buf, tv, need, mref):
    li = lax.iota(jnp.int32, 16)
    zi = jnp.zeros((16,), jnp.int32)
    nch = (cn + 15) // 16

    def it(i, carry):
        tie, cur = carry
        idxv = cref[pl.ds(i * 16, 16)]
        idxg = jnp.minimum(jnp.maximum(idxv, 0), N - 1)
        vals = plsc.load_gather(buf, [idxg])
        valid = (li + i * 16) < cn
        mg = valid & (vals > tv)
        me = valid & (vals == tv)
        ime = me.astype(jnp.int32)
        pe = plsc.cumsum(ime) - ime
        mm = mg | (me & (tie + pe < need))
        imm = mm.astype(jnp.int32)
        pos = cur + plsc.cumsum(imm) - imm
        plsc.store_scatter(mref, [pos], idxv, mask=mm)
        return (tie + jnp.sum(ime), cur + plsc.all_reduce_population_count(mm))

    lax.fori_loop(0, nch, it, (jnp.int32(0), zi))


def _membership_full(tb, lb, tvt, needt, tvl, needl, tmem):
    li = lax.iota(jnp.int32, 16)
    zero = jnp.int32(0)
    zi = jnp.zeros((16,), jnp.int32)

    def chunk(t, x, iv, c):
        tieT, tieL, cur, ovv = c
        mTg = t > tvt
        mTe = t == tvt
        mLg = x > tvl
        mLe = x == tvl
        iTe = mTe.astype(jnp.int32)
        iLe = mLe.astype(jnp.int32)
        peT = plsc.cumsum(iTe) - iTe
        peL = plsc.cumsum(iLe) - iLe
        memT = mTg | (mTe & (tieT + peT < needt))
        memL = mLg | (mLe & (tieL + peL < needl))
        imT = memT.astype(jnp.int32)
        pos = cur + plsc.cumsum(imT) - imT
        plsc.store_scatter(tmem, [pos], iv, mask=memT)
        return (tieT + jnp.sum(iTe),
                tieL + jnp.sum(iLe),
                cur + plsc.all_reduce_population_count(memT),
                ovv + (memT & memL).astype(jnp.int32))

    def it(g, carry):
        base = g * (16 * UNROLL)
        ts = [tb[pl.ds(base + 16 * j, 16)] for j in range(UNROLL)]
        xs = [lb[pl.ds(base + 16 * j, 16)] for j in range(UNROLL)]
        tmax = jnp.maximum(jnp.maximum(ts[0], ts[1]), jnp.maximum(ts[2], ts[3]))
        xmax = jnp.maximum(jnp.maximum(xs[0], xs[1]), jnp.maximum(xs[2], xs[3]))
        hit = _any((tmax >= tvt) | (xmax >= tvl))

        def slow(c):
            for j in range(UNROLL):
                c = chunk(ts[j], xs[j], li + (base + 16 * j), c)
            return c

        return lax.cond(hit, slow, lambda c: c, carry)

    carry = lax.fori_loop(0, NGROUP, it, (zero, zero, zi, zi))
    return jnp.sum(carry[3])


def _neg_log_sigmoid(x):
    s = 1.0 / (1.0 + jnp.exp(-x))
    y = s + jnp.float32(EPS)
    bits = plsc.bitcast(y, jnp.int32)
    e = (bits >> 23) - 127
    m = plsc.bitcast((bits & 0x7FFFFF) | 0x3F800000, jnp.float32)
    z = (m - 1.0) / (m + 1.0)
    z2 = z * z
    p = 1.0 + z2 * (jnp.float32(1 / 3) + z2 * (jnp.float32(1 / 5)
          + z2 * (jnp.float32(1 / 7) + z2 * jnp.float32(1 / 9))))
    lny = e.astype(jnp.float32) * jnp.float32(LN2) + 2.0 * z * p
    return -lny


def _sc_body(logits_hbm, targets_hbm, out_hbm,
             tbuf, lbuf, candT, candL, tmem, lmem, obuf, semt, seml):
    cid = lax.axis_index("c")
    sid = lax.axis_index("s")
    wid = sid * 2 + cid

    li = lax.iota(jnp.int32, 16)

    def row_it(r, lossvec):
        row = wid * 4 + r
        ct = pltpu.async_copy(targets_hbm.at[row], tbuf, semt)
        cl = pltpu.async_copy(logits_hbm.at[row], lbuf, seml)
        ct.wait()
        cl.wait()
        tmem[pl.ds(0, 16)] = jnp.full((16,), -1, jnp.int32)
        tmem[pl.ds(16, 16)] = jnp.full((16,), -1, jnp.int32)
        lmem[pl.ds(0, 16)] = jnp.full((16,), -2, jnp.int32)
        lmem[pl.ds(16, 16)] = jnp.full((16,), -2, jnp.int32)

        cnt, cnl = _scan_collect2(tbuf, lbuf, candT, candL)
        overflow = (cnt > CAP - 1) | (cnl > CAP - 1)

        def fast(_):
            tvt, needt = _select30(candT, cnt, tbuf)
            tvl, needl = _select30(candL, cnl, lbuf)
            _cand_members(candT, cnt, tbuf, tvt, needt, tmem)
            _cand_members(candL, cnl, lbuf, tvl, needl, lmem)
            t0 = tmem[pl.ds(0, 16)]
            t1 = tmem[pl.ds(16, 16)]
            acc = jnp.zeros((16,), jnp.int32)
            for sh in range(16):
                perm = (li + sh) & 15
                r0 = plsc.load_gather(lmem, [perm])
                r1 = plsc.load_gather(lmem, [perm + 16])
                acc = (acc + (t0 == r0).astype(jnp.int32)
                       + (t0 == r1).astype(jnp.int32)
                       + (t1 == r0).astype(jnp.int32)
                       + (t1 == r1).astype(jnp.int32))
            return jnp.sum(acc)

        def slowfb(_):
            tvt, needt, tvl, needl = _scan_topk2_full(tbuf, lbuf)
            return _membership_full(tbuf, lbuf, tvt, needt, tvl, needl, tmem)

        ov = lax.cond(overflow, slowfb, fast, None)

        t0 = jnp.maximum(tmem[pl.ds(0, 16)], 0)
        t1 = jnp.maximum(tmem[pl.ds(16, 16)], 0)
        g0 = plsc.load_gather(lbuf, [t0])
        g1 = plsc.load_gather(lbuf, [t1])
        f0 = _neg_log_sigmoid(g0)
        f1 = jnp.where(li < K - 16, _neg_log_sigmoid(g1), 0.0)
        fsum = jnp.sum(f0 + f1)
        w = 1.0 - ov.astype(jnp.float32) * jnp.float32(1.0 / K)
        loss_r = fsum * jnp.float32(1.0 / K) * w
        return jnp.where(li == r, loss_r, lossvec)

    lossvec = lax.fori_loop(0, 4, row_it, jnp.zeros((16,), jnp.float32))
    obuf[...] = lossvec
    pltpu.sync_copy(obuf, out_hbm.at[wid])


@jax.jit
def _sc_call(logits, targets):
    fn = functools.partial(
        pl.kernel,
        out_type=jax.ShapeDtypeStruct((32, 16), jnp.float32),
        mesh=plsc.VectorSubcoreMesh(core_axis_name="c", subcore_axis_name="s"),
        compiler_params=pltpu.CompilerParams(needs_layout_passes=False),
        scratch_types=[
            pltpu.VMEM((N,), jnp.float32),
            pltpu.VMEM((N,), jnp.float32),
            pltpu.VMEM((CAP,), jnp.int32),
            pltpu.VMEM((CAP,), jnp.int32),
            pltpu.VMEM((32,), jnp.int32),
            pltpu.VMEM((32,), jnp.int32),
            pltpu.VMEM((16,), jnp.float32),
            pltpu.SemaphoreType.DMA,
            pltpu.SemaphoreType.DMA,
        ],
    )(_sc_body)
    part = fn(logits, targets)
    return jnp.sum(part) * jnp.float32(1.0 / B)


def kernel(logits, targets):
    return _sc_call(logits, targets)
